# Initial kernel scaffold; baseline (speedup 1.0000x reference)
import jax
import jax.numpy as jnp
from jax.experimental import pallas as pl


def kernel(raw_x, w_real, w_imag, W1, b1, W2, b2, W_ih, W_hh, b_ih, b_hh, W_cls, b_cls):
    raise NotImplementedError("write your pallas kernel here")


# speedup vs baseline: 8.3519x; 8.3519x over previous
import math

import jax
import jax.numpy as jnp
from jax.experimental import pallas as pl

B, C, T, F, H, M, NCLS, STRIDE = 4, 16, 128, 16, 64, 64, 4, 8
S = T // STRIDE
THETA = math.radians(45.0)
E = C * (C - 1)
GATE_COUNT = float(B * E * F * S)


def _core(ws_r_ref, ws_i_ref, raw_ref, wm_ref, wa_ref, wsr_ref, wdr_ref,
          w1sT_ref, w1dT_ref, b1_ref, w2T_ref, b2_ref,
          wihT_ref, whhT_ref, bih_ref, bhh_ref, wclsT_ref, bcls_ref,
          logits_ref, rate_ref):
    f32 = jnp.float32
    bf16 = jnp.bfloat16
    row = jax.lax.broadcasted_iota(jnp.int32, (C, C), 0)
    col = jax.lax.broadcasted_iota(jnp.int32, (C, C), 1)
    offdiag = (row != col).astype(f32)

    wm = wm_ref[...]
    wa = wa_ref[...]
    wsr = wsr_ref[...]
    wdr = wdr_ref[...]
    b1 = b1_ref[...]
    b2 = b2_ref[...]

    def step(s, carry):
        state, gsum = carry
        wr = ws_r_ref[s]
        wi = ws_i_ref[s]
        raw = raw_ref[s]

        a_r = wr[:, :, None, :]
        a_i = wi[:, :, None, :]
        b_r = wr[:, None, :, :]
        b_i = wi[:, None, :, :]
        x_re = a_r * b_r + a_i * b_i
        x_im = a_i * b_r - a_r * b_i
        mag = jnp.sqrt(x_re * x_re + x_im * x_im + 1e-12)
        ang = jnp.arctan2(x_im, x_re)
        delta = jnp.arctan2(jnp.sin(ang), jnp.cos(ang))
        gate = (delta > THETA).astype(f32)
        gate = jnp.nan_to_num(gate, nan=0.0, posinf=0.0, neginf=0.0)
        mag = jnp.nan_to_num(mag, nan=0.0, posinf=0.0, neginf=0.0)
        ang = jnp.nan_to_num(ang, nan=0.0, posinf=0.0, neginf=0.0)
        gate = gate * offdiag[None, :, :, None]
        gsum = gsum + jnp.sum(gate)

        mag = mag.astype(bf16).astype(f32)
        ang = ang.astype(bf16).astype(f32)
        raw = raw.astype(bf16).astype(f32)
        state_b = state.astype(bf16)
        s_src = jnp.dot(state_b, w1sT_ref[...], preferred_element_type=f32)
        s_dst = jnp.dot(state_b, w1dT_ref[...], preferred_element_type=f32)
        p_src = raw[:, :, None] * wsr[None] + s_src.reshape(B, C, M)
        p_dst = raw[:, :, None] * wdr[None] + s_dst.reshape(B, C, M)
        const = p_src[:, :, None, :] + p_dst[:, None, :, :] + b1[None, None]

        pre = (mag[..., None] * wm[None, None] + ang[..., None] * wa[None, None]
               + const[:, :, :, None, :])
        h1 = jnp.maximum(pre, 0.0).astype(bf16).astype(f32)
        acc = jnp.sum(h1 * gate[..., None], axis=3)
        gh = jnp.sum(acc, axis=1).reshape(B * C, M)
        gcnt = jnp.sum(gate, axis=(1, 3)).reshape(B * C, 1)
        gh_hi = gh.astype(bf16)
        gh_lo = (gh - gh_hi.astype(f32)).astype(bf16)
        agg = (jnp.dot(gh_hi, w2T_ref[...], preferred_element_type=f32)
               + jnp.dot(gh_lo, w2T_ref[...], preferred_element_type=f32)
               + b2 * gcnt)

        gi = jnp.dot(agg.astype(bf16), wihT_ref[...], preferred_element_type=f32) + bih_ref[...]
        gg = jnp.dot(state_b, whhT_ref[...], preferred_element_type=f32) + bhh_ref[...]
        r = jax.nn.sigmoid(gi[:, :H] + gg[:, :H])
        z = jax.nn.sigmoid(gi[:, H:2 * H] + gg[:, H:2 * H])
        n = jnp.tanh(gi[:, 2 * H:] + r * gg[:, 2 * H:])
        state = (1.0 - z) * n + z * state
        return state, gsum

    state0 = jnp.zeros((B * C, H), dtype=f32)
    gsum0 = jnp.zeros((1, 1), dtype=f32)
    state, gsum = jax.lax.fori_loop(0, S, step, (state0, gsum0))
    pooled = jnp.mean(state.reshape(B, C, H), axis=1)
    logits = jnp.dot(pooled.astype(bf16), wclsT_ref[...], preferred_element_type=f32) + bcls_ref[...]
    logits_ref[...] = logits
    rate_ref[...] = gsum / GATE_COUNT


def kernel(raw_x, w_real, w_imag, W1, b1, W2, b2, W_ih, W_hh, b_ih, b_hh, W_cls, b_cls):
    f32 = jnp.float32
    ws_r = jnp.transpose(w_real[:, :, ::STRIDE, :], (2, 0, 1, 3))
    ws_i = jnp.transpose(w_imag[:, :, ::STRIDE, :], (2, 0, 1, 3))
    raws = jnp.transpose(raw_x[:, :, ::STRIDE], (2, 0, 1))

    bf16 = jnp.bfloat16
    W1b = W1.astype(bf16).astype(f32)
    wm = W1b[:, 0].reshape(1, M)
    wa = W1b[:, 1].reshape(1, M)
    wsr = W1b[:, 2].reshape(1, M)
    wdr = W1b[:, 3].reshape(1, M)
    w1sT = jnp.transpose(W1[:, 4:4 + H]).astype(bf16)
    w1dT = jnp.transpose(W1[:, 4 + H:4 + 2 * H]).astype(bf16)

    logits, rate = pl.pallas_call(
        _core,
        out_shape=[jax.ShapeDtypeStruct((B, NCLS), f32),
                   jax.ShapeDtypeStruct((1, 1), f32)],
    )(ws_r, ws_i, raws, wm, wa, wsr, wdr, w1sT, w1dT,
      b1.reshape(1, M), jnp.transpose(W2).astype(bf16), b2.reshape(1, H),
      jnp.transpose(W_ih).astype(bf16), jnp.transpose(W_hh).astype(bf16),
      b_ih.reshape(1, 3 * H), b_hh.reshape(1, 3 * H),
      jnp.transpose(W_cls).astype(bf16), b_cls.reshape(1, NCLS))
    return logits, rate[0, 0]
